# Initial kernel scaffold; baseline (speedup 1.0000x reference)
import jax
import jax.numpy as jnp
from jax.experimental import pallas as pl


def kernel(x, edge_index, W1, b1, W2, b2, Wc1, bc1, Wc2, bc2, Wc3, bc3):
    raise NotImplementedError("write your pallas kernel here")


# speedup vs baseline: 2.9660x; 2.9660x over previous
import functools

import jax
import jax.numpy as jnp
from jax import lax
from jax.experimental import pallas as pl
from jax.experimental.pallas import tpu as pltpu
from jax.experimental.pallas import tpu_sc as plsc

N = 10000
E = 320000
D = 128
H = 128
DH = D // 2

NC = 2
NS = 16
EPT = E // NS
K = 80
NITER = EPT // K
KV = K // 16

RPT = 624
RTAIL = N - NS * RPT

_MESH = plsc.VectorSubcoreMesh(core_axis_name="c", subcore_axis_name="s")


@functools.partial(
    pl.kernel,
    out_type=jax.ShapeDtypeStruct((NC * 2 * N,), jnp.float32),
    mesh=_MESH,
    scratch_types=[
        pltpu.VMEM((K,), jnp.int32),
        pltpu.VMEM((K,), jnp.float32),
        pltpu.VMEM((RPT,), jnp.float32),
        pltpu.VMEM_SHARED((N,), jnp.float32),
        pltpu.VMEM_SHARED((N,), jnp.float32),
        pltpu.SemaphoreType.DMA,
    ],
)
def _deg_kernel(src_hbm, dst_hbm, zeros1_hbm, ones_hbm, out_hbm,
                idx_v, ones_v, stage_v, acc_src, acc_dst, sem):
    cid = lax.axis_index("c")
    sid = lax.axis_index("s")
    wid = cid * NS + sid
    base0 = wid * (E // (NC * NS))

    pltpu.sync_copy(zeros1_hbm, stage_v)
    pltpu.sync_copy(stage_v, acc_src.at[pl.ds(sid * RPT, RPT)])
    pltpu.sync_copy(stage_v, acc_dst.at[pl.ds(sid * RPT, RPT)])

    @pl.when(sid == NS - 1)
    def _zero_tail():
        pltpu.sync_copy(stage_v.at[pl.ds(0, RTAIL)],
                        acc_src.at[pl.ds(NS * RPT, RTAIL)])
        pltpu.sync_copy(stage_v.at[pl.ds(0, RTAIL)],
                        acc_dst.at[pl.ds(NS * RPT, RTAIL)])

    pltpu.sync_copy(ones_hbm, ones_v)
    plsc.subcore_barrier()

    def body(i, carry):
        base = base0 + i * K
        pltpu.sync_copy(src_hbm.at[pl.ds(base, K)], idx_v)
        pltpu.sync_copy(ones_v, acc_src.at[idx_v], add=True)
        pltpu.sync_copy(dst_hbm.at[pl.ds(base, K)], idx_v)
        pltpu.sync_copy(ones_v, acc_dst.at[idx_v], add=True)
        return carry

    lax.fori_loop(0, E // (NC * NS) // K, body, 0)
    plsc.subcore_barrier()

    obase = cid * 2 * N
    pltpu.sync_copy(acc_src.at[pl.ds(sid * RPT, RPT)], stage_v)
    pltpu.sync_copy(stage_v, out_hbm.at[pl.ds(obase + sid * RPT, RPT)])
    pltpu.sync_copy(acc_dst.at[pl.ds(sid * RPT, RPT)], stage_v)
    pltpu.sync_copy(stage_v, out_hbm.at[pl.ds(obase + N + sid * RPT, RPT)])

    @pl.when(sid == NS - 1)
    def _write_tail():
        pltpu.sync_copy(acc_src.at[pl.ds(NS * RPT, RTAIL)],
                        stage_v.at[pl.ds(0, RTAIL)])
        pltpu.sync_copy(stage_v.at[pl.ds(0, RTAIL)],
                        out_hbm.at[pl.ds(obase + NS * RPT, RTAIL)])
        pltpu.sync_copy(acc_dst.at[pl.ds(NS * RPT, RTAIL)],
                        stage_v.at[pl.ds(0, RTAIL)])
        pltpu.sync_copy(stage_v.at[pl.ds(0, RTAIL)],
                        out_hbm.at[pl.ds(obase + N + NS * RPT, RTAIL)])


@functools.partial(
    pl.kernel,
    out_type=jax.ShapeDtypeStruct((NC, N, DH), jnp.float32),
    mesh=_MESH,
    scratch_types=[
        pltpu.VMEM((K,), jnp.int32),
        pltpu.VMEM((K,), jnp.int32),
        pltpu.VMEM((K, DH), jnp.float32),
        pltpu.VMEM((RPT, DH), jnp.float32),
        pltpu.VMEM_SHARED((N, DH), jnp.float32),
        pltpu.SemaphoreType.DMA,
    ],
    compiler_params=pltpu.CompilerParams(use_tc_tiling_on_sc=False),
)
def _agg_kernel(feat_hbm, src_hbm, dst_hbm, zeros2_hbm, out_hbm,
                src_v, dst_v, rows_v, stage_v, acc_sh, sem):
    cid = lax.axis_index("c")
    sid = lax.axis_index("s")
    base0 = sid * EPT
    coff = cid * N

    pltpu.sync_copy(zeros2_hbm, stage_v)
    pltpu.sync_copy(stage_v, acc_sh.at[pl.ds(sid * RPT, RPT)])

    @pl.when(sid == NS - 1)
    def _zero_tail():
        pltpu.sync_copy(stage_v.at[pl.ds(0, RTAIL)],
                        acc_sh.at[pl.ds(NS * RPT, RTAIL)])

    plsc.subcore_barrier()

    def body(i, carry):
        base = base0 + i * K
        pltpu.sync_copy(src_hbm.at[pl.ds(base, K)], src_v)
        pltpu.sync_copy(dst_hbm.at[pl.ds(base, K)], dst_v)
        for j in range(KV):
            src_v[pl.ds(j * 16, 16)] = src_v[pl.ds(j * 16, 16)] + coff
        pltpu.async_copy(feat_hbm.at[src_v], rows_v, sem).wait()
        pltpu.sync_copy(rows_v, acc_sh.at[dst_v], add=True)
        return carry

    lax.fori_loop(0, NITER, body, 0)
    plsc.subcore_barrier()

    pltpu.sync_copy(acc_sh.at[pl.ds(sid * RPT, RPT)], stage_v)
    pltpu.sync_copy(stage_v, out_hbm.at[cid, pl.ds(sid * RPT, RPT)])

    @pl.when(sid == NS - 1)
    def _write_tail():
        pltpu.sync_copy(acc_sh.at[pl.ds(NS * RPT, RTAIL)],
                        stage_v.at[pl.ds(0, RTAIL)])
        pltpu.sync_copy(stage_v.at[pl.ds(0, RTAIL)],
                        out_hbm.at[cid, pl.ds(NS * RPT, RTAIL)])


RB = 1000
GRID = N // RB


def _norm_body(deg_ref, x_ref, xp_ref, ns_ref, nd_ref):
    deg = deg_ref[...]
    dsrc = deg[:, 0:1] + deg[:, 2:3]
    ddst = deg[:, 1:2] + deg[:, 3:4]
    ns = lax.rsqrt(jnp.where(dsrc > 0, dsrc, 1.0))
    nd = lax.rsqrt(jnp.where(ddst > 0, ddst, 1.0))
    xp = x_ref[...] * ns
    xp_ref[0] = xp[:, :DH]
    xp_ref[1] = xp[:, DH:]
    ns_ref[...] = ns
    nd_ref[...] = nd


def _layer1_body(a0_ref, a1_ref, wa_ref, wb_ref, b_ref, nd_ref, ns_ref,
                 out_ref):
    h = (jnp.dot(a0_ref[0], wa_ref[...], preferred_element_type=jnp.float32, precision=lax.Precision.HIGHEST)
         + jnp.dot(a1_ref[0], wb_ref[...], preferred_element_type=jnp.float32, precision=lax.Precision.HIGHEST))
    h = jnp.maximum(h * nd_ref[...] + b_ref[...], 0.0)
    h = h * ns_ref[...]
    out_ref[0] = h[:, :DH]
    out_ref[1] = h[:, DH:]


def _final_body(a0_ref, a1_ref, wa_ref, wb_ref, b_ref, nd_ref,
                wc1_ref, bc1_ref, wc2_ref, bc2_ref, wc3_ref, bc3_ref,
                out_ref, acc_ref):
    i = pl.program_id(0)

    @pl.when(i == 0)
    def _init():
        acc_ref[...] = jnp.zeros_like(acc_ref)

    h = (jnp.dot(a0_ref[0], wa_ref[...], preferred_element_type=jnp.float32, precision=lax.Precision.HIGHEST)
         + jnp.dot(a1_ref[0], wb_ref[...], preferred_element_type=jnp.float32, precision=lax.Precision.HIGHEST))
    h = jnp.maximum(h * nd_ref[...] + b_ref[...], 0.0)
    acc_ref[...] += jnp.sum(h, axis=0, keepdims=True)

    @pl.when(i == pl.num_programs(0) - 1)
    def _head():
        hg = acc_ref[...] * (1.0 / N)
        o = jnp.dot(hg, wc1_ref[...], preferred_element_type=jnp.float32, precision=lax.Precision.HIGHEST)
        o = jnp.maximum(o + bc1_ref[...], 0.0)
        o = jnp.dot(o, wc2_ref[...], preferred_element_type=jnp.float32, precision=lax.Precision.HIGHEST)
        o = jnp.maximum(o + bc2_ref[...], 0.0)
        out_ref[...] = (jnp.dot(o, wc3_ref[...], preferred_element_type=jnp.float32, precision=lax.Precision.HIGHEST)
                        + bc3_ref[...])


def kernel(x, edge_index, W1, b1, W2, b2, Wc1, bc1, Wc2, bc2, Wc3, bc3):
    edge_index = edge_index.astype(jnp.int32)
    src = edge_index[0]
    dst = edge_index[1]
    zeros1 = jnp.zeros((RPT,), jnp.float32)
    zeros2 = jnp.zeros((RPT, DH), jnp.float32)
    ones_k = jnp.ones((K,), jnp.float32)

    deg = _deg_kernel(src, dst, zeros1, ones_k)
    degT = deg.reshape(2 * NC, N).T

    xp, nsrc, ndst = pl.pallas_call(
        _norm_body,
        grid=(GRID,),
        in_specs=[
            pl.BlockSpec((RB, 2 * NC), lambda i: (i, 0)),
            pl.BlockSpec((RB, D), lambda i: (i, 0)),
        ],
        out_specs=[
            pl.BlockSpec((2, RB, DH), lambda i: (0, i, 0)),
            pl.BlockSpec((RB, 1), lambda i: (i, 0)),
            pl.BlockSpec((RB, 1), lambda i: (i, 0)),
        ],
        out_shape=[
            jax.ShapeDtypeStruct((2, N, DH), jnp.float32),
            jax.ShapeDtypeStruct((N, 1), jnp.float32),
            jax.ShapeDtypeStruct((N, 1), jnp.float32),
        ],
    )(degT, x)

    agg1 = _agg_kernel(xp.reshape(2 * N, DH), src, dst, zeros2)

    h1p = pl.pallas_call(
        _layer1_body,
        grid=(GRID,),
        in_specs=[
            pl.BlockSpec((1, RB, DH), lambda i: (0, i, 0)),
            pl.BlockSpec((1, RB, DH), lambda i: (1, i, 0)),
            pl.BlockSpec((DH, H), lambda i: (0, 0)),
            pl.BlockSpec((DH, H), lambda i: (1, 0)),
            pl.BlockSpec((1, H), lambda i: (0, 0)),
            pl.BlockSpec((RB, 1), lambda i: (i, 0)),
            pl.BlockSpec((RB, 1), lambda i: (i, 0)),
        ],
        out_specs=pl.BlockSpec((2, RB, DH), lambda i: (0, i, 0)),
        out_shape=jax.ShapeDtypeStruct((2, N, DH), jnp.float32),
    )(agg1, agg1, W1, W1, b1.reshape(1, H), ndst, nsrc)

    agg2 = _agg_kernel(h1p.reshape(2 * N, DH), src, dst, zeros2)

    out = pl.pallas_call(
        _final_body,
        grid=(GRID,),
        in_specs=[
            pl.BlockSpec((1, RB, DH), lambda i: (0, i, 0)),
            pl.BlockSpec((1, RB, DH), lambda i: (1, i, 0)),
            pl.BlockSpec((DH, H), lambda i: (0, 0)),
            pl.BlockSpec((DH, H), lambda i: (1, 0)),
            pl.BlockSpec((1, H), lambda i: (0, 0)),
            pl.BlockSpec((RB, 1), lambda i: (i, 0)),
            pl.BlockSpec((H, H), lambda i: (0, 0)),
            pl.BlockSpec((1, H), lambda i: (0, 0)),
            pl.BlockSpec((H, H), lambda i: (0, 0)),
            pl.BlockSpec((1, H), lambda i: (0, 0)),
            pl.BlockSpec((H, 1), lambda i: (0, 0)),
            pl.BlockSpec((1, 1), lambda i: (0, 0)),
        ],
        out_specs=pl.BlockSpec((1, 1), lambda i: (0, 0)),
        out_shape=jax.ShapeDtypeStruct((1, 1), jnp.float32),
        scratch_shapes=[pltpu.VMEM((1, H), jnp.float32)],
    )(agg2, agg2, W2, W2, b2.reshape(1, H), ndst,
      Wc1, bc1.reshape(1, H), Wc2, bc2.reshape(1, H),
      Wc3, bc3.reshape(1, 1))

    return out


# speedup vs baseline: 8.8617x; 2.9878x over previous
import functools

import jax
import jax.numpy as jnp
from jax import lax
from jax.experimental import pallas as pl
from jax.experimental.pallas import tpu as pltpu
from jax.experimental.pallas import tpu_sc as plsc

N = 10000
E = 320000
D = 128
H = 128
DH = D // 2

NC = 2
NS = 16
EPT = E // NS
K = 125
NITER = EPT // K
SEG = 4
SROWS = NITER // SEG
DNITER = E // (NC * NS) // K
EROWS = E // K

RPT = 624
RTAIL = N - NS * RPT

_MESH = plsc.VectorSubcoreMesh(core_axis_name="c", subcore_axis_name="s")


@functools.partial(
    pl.kernel,
    out_type=jax.ShapeDtypeStruct((NC * 2 * N,), jnp.float32),
    mesh=_MESH,
    scratch_types=[
        pltpu.VMEM((DNITER, K), jnp.int32),
        pltpu.VMEM((DNITER, K), jnp.int32),
        pltpu.VMEM((K,), jnp.float32),
        pltpu.VMEM((RPT,), jnp.float32),
        pltpu.VMEM_SHARED((N,), jnp.float32),
        pltpu.VMEM_SHARED((N,), jnp.float32),
        pltpu.SemaphoreType.DMA,
    ],
    compiler_params=pltpu.CompilerParams(use_tc_tiling_on_sc=False),
)
def _deg_kernel(src_hbm, dst_hbm, zeros1_hbm, ones_hbm, out_hbm,
                src_all, dst_all, ones_v, stage_v, acc_src, acc_dst, sem):
    cid = lax.axis_index("c")
    sid = lax.axis_index("s")
    wid = cid * NS + sid

    pltpu.sync_copy(src_hbm.at[pl.ds(wid * DNITER, DNITER)], src_all)
    pltpu.sync_copy(dst_hbm.at[pl.ds(wid * DNITER, DNITER)], dst_all)

    pltpu.sync_copy(zeros1_hbm, stage_v)
    pltpu.sync_copy(stage_v, acc_src.at[pl.ds(sid * RPT, RPT)])
    pltpu.sync_copy(stage_v, acc_dst.at[pl.ds(sid * RPT, RPT)])

    @pl.when(sid == NS - 1)
    def _zero_tail():
        pltpu.sync_copy(stage_v.at[pl.ds(0, RTAIL)],
                        acc_src.at[pl.ds(NS * RPT, RTAIL)])
        pltpu.sync_copy(stage_v.at[pl.ds(0, RTAIL)],
                        acc_dst.at[pl.ds(NS * RPT, RTAIL)])

    pltpu.sync_copy(ones_hbm, ones_v)
    plsc.subcore_barrier()

    def body(i, carry):
        pltpu.sync_copy(ones_v, acc_src.at[src_all.at[i]], add=True)
        pltpu.sync_copy(ones_v, acc_dst.at[dst_all.at[i]], add=True)
        return carry

    lax.fori_loop(0, DNITER, body, 0)
    plsc.subcore_barrier()

    obase = cid * 2 * N
    pltpu.sync_copy(acc_src.at[pl.ds(sid * RPT, RPT)], stage_v)
    pltpu.sync_copy(stage_v, out_hbm.at[pl.ds(obase + sid * RPT, RPT)])
    pltpu.sync_copy(acc_dst.at[pl.ds(sid * RPT, RPT)], stage_v)
    pltpu.sync_copy(stage_v, out_hbm.at[pl.ds(obase + N + sid * RPT, RPT)])

    @pl.when(sid == NS - 1)
    def _write_tail():
        pltpu.sync_copy(acc_src.at[pl.ds(NS * RPT, RTAIL)],
                        stage_v.at[pl.ds(0, RTAIL)])
        pltpu.sync_copy(stage_v.at[pl.ds(0, RTAIL)],
                        out_hbm.at[pl.ds(obase + NS * RPT, RTAIL)])
        pltpu.sync_copy(acc_dst.at[pl.ds(NS * RPT, RTAIL)],
                        stage_v.at[pl.ds(0, RTAIL)])
        pltpu.sync_copy(stage_v.at[pl.ds(0, RTAIL)],
                        out_hbm.at[pl.ds(obase + N + NS * RPT, RTAIL)])


@functools.partial(
    pl.kernel,
    out_type=jax.ShapeDtypeStruct((NC, N, DH), jnp.float32),
    mesh=_MESH,
    scratch_types=[
        pltpu.VMEM((SROWS, K), jnp.int32),
        pltpu.VMEM((SROWS, K), jnp.int32),
        pltpu.VMEM((K, DH), jnp.float32),
        pltpu.VMEM((K, DH), jnp.float32),
        pltpu.VMEM_SHARED((N, DH), jnp.float32),
        pltpu.SemaphoreType.DMA,
        pltpu.SemaphoreType.DMA,
    ],
    compiler_params=pltpu.CompilerParams(use_tc_tiling_on_sc=False),
)
def _agg_kernel(feat_hbm, src_hbm, dst_hbm, zeros2_hbm, out_hbm,
                src_seg, dst_seg, rows0, rows1, acc_sh, sem0, sem1):
    cid = lax.axis_index("c")
    sid = lax.axis_index("s")
    feat = feat_hbm.at[pl.ds(cid * N, N)]

    pltpu.sync_copy(zeros2_hbm, rows0)
    for j in range(RPT // K):
        pltpu.sync_copy(rows0, acc_sh.at[pl.ds(sid * RPT + j * K, K)])
    pltpu.sync_copy(rows0.at[pl.ds(0, RPT % K)],
                    acc_sh.at[pl.ds(sid * RPT + (RPT // K) * K, RPT % K)])

    @pl.when(sid == NS - 1)
    def _zero_tail():
        pltpu.sync_copy(rows0.at[pl.ds(0, RTAIL)],
                        acc_sh.at[pl.ds(NS * RPT, RTAIL)])

    plsc.subcore_barrier()

    bufs = (rows0, rows1)
    sems = (sem0, sem1)

    def seg_body(s, carry):
        base_row = sid * NITER + s * SROWS
        pltpu.sync_copy(src_hbm.at[pl.ds(base_row, SROWS)], src_seg)
        pltpu.sync_copy(dst_hbm.at[pl.ds(base_row, SROWS)], dst_seg)
        pltpu.async_copy(feat.at[src_seg.at[0]], rows0, sem0)
        pltpu.async_copy(feat.at[src_seg.at[1]], rows1, sem1)

        def body(g, c):
            for b in range(2):
                i = g * 2 + b
                pltpu.make_async_copy(feat.at[src_seg.at[i]], bufs[b],
                                      sems[b]).wait()
                pltpu.sync_copy(bufs[b], acc_sh.at[dst_seg.at[i]], add=True)

                @pl.when(i + 2 < SROWS)
                def _prefetch():
                    pltpu.async_copy(feat.at[src_seg.at[i + 2]], bufs[b],
                                     sems[b])
            return c

        lax.fori_loop(0, SROWS // 2, body, 0)
        return carry

    lax.fori_loop(0, SEG, seg_body, 0)
    plsc.subcore_barrier()

    for j in range(RPT // K):
        pltpu.sync_copy(acc_sh.at[pl.ds(sid * RPT + j * K, K)], rows0)
        pltpu.sync_copy(rows0, out_hbm.at[cid, pl.ds(sid * RPT + j * K, K)])
    pltpu.sync_copy(acc_sh.at[pl.ds(sid * RPT + (RPT // K) * K, RPT % K)],
                    rows0.at[pl.ds(0, RPT % K)])
    pltpu.sync_copy(rows0.at[pl.ds(0, RPT % K)],
                    out_hbm.at[cid, pl.ds(sid * RPT + (RPT // K) * K, RPT % K)])

    @pl.when(sid == NS - 1)
    def _write_tail():
        pltpu.sync_copy(acc_sh.at[pl.ds(NS * RPT, RTAIL)],
                        rows1.at[pl.ds(0, RTAIL)])
        pltpu.sync_copy(rows1.at[pl.ds(0, RTAIL)],
                        out_hbm.at[cid, pl.ds(NS * RPT, RTAIL)])


RB = 1000
GRID = N // RB


def _norm_body(deg_ref, x_ref, xp_ref, ns_ref, nd_ref):
    deg = deg_ref[...]
    dsrc = deg[:, 0:1] + deg[:, 2:3]
    ddst = deg[:, 1:2] + deg[:, 3:4]
    ns = lax.rsqrt(jnp.where(dsrc > 0, dsrc, 1.0))
    nd = lax.rsqrt(jnp.where(ddst > 0, ddst, 1.0))
    xp = x_ref[...] * ns
    xp_ref[0] = xp[:, :DH]
    xp_ref[1] = xp[:, DH:]
    ns_ref[...] = ns
    nd_ref[...] = nd


def _layer1_body(a0_ref, a1_ref, wa_ref, wb_ref, b_ref, nd_ref, ns_ref,
                 out_ref):
    h = (jnp.dot(a0_ref[0], wa_ref[...], preferred_element_type=jnp.float32, precision=lax.Precision.HIGHEST)
         + jnp.dot(a1_ref[0], wb_ref[...], preferred_element_type=jnp.float32, precision=lax.Precision.HIGHEST))
    h = jnp.maximum(h * nd_ref[...] + b_ref[...], 0.0)
    h = h * ns_ref[...]
    out_ref[0] = h[:, :DH]
    out_ref[1] = h[:, DH:]


def _final_body(a0_ref, a1_ref, wa_ref, wb_ref, b_ref, nd_ref,
                wc1_ref, bc1_ref, wc2_ref, bc2_ref, wc3_ref, bc3_ref,
                out_ref, acc_ref):
    i = pl.program_id(0)

    @pl.when(i == 0)
    def _init():
        acc_ref[...] = jnp.zeros_like(acc_ref)

    h = (jnp.dot(a0_ref[0], wa_ref[...], preferred_element_type=jnp.float32, precision=lax.Precision.HIGHEST)
         + jnp.dot(a1_ref[0], wb_ref[...], preferred_element_type=jnp.float32, precision=lax.Precision.HIGHEST))
    h = jnp.maximum(h * nd_ref[...] + b_ref[...], 0.0)
    acc_ref[...] += jnp.sum(h, axis=0, keepdims=True)

    @pl.when(i == pl.num_programs(0) - 1)
    def _head():
        hg = acc_ref[...] * (1.0 / N)
        o = jnp.dot(hg, wc1_ref[...], preferred_element_type=jnp.float32, precision=lax.Precision.HIGHEST)
        o = jnp.maximum(o + bc1_ref[...], 0.0)
        o = jnp.dot(o, wc2_ref[...], preferred_element_type=jnp.float32, precision=lax.Precision.HIGHEST)
        o = jnp.maximum(o + bc2_ref[...], 0.0)
        out_ref[...] = (jnp.dot(o, wc3_ref[...], preferred_element_type=jnp.float32, precision=lax.Precision.HIGHEST)
                        + bc3_ref[...])


def kernel(x, edge_index, W1, b1, W2, b2, Wc1, bc1, Wc2, bc2, Wc3, bc3):
    edge_index = edge_index.astype(jnp.int32)
    src = edge_index[0].reshape(EROWS, K)
    dst = edge_index[1].reshape(EROWS, K)
    zeros1 = jnp.zeros((RPT,), jnp.float32)
    zeros2 = jnp.zeros((K, DH), jnp.float32)
    ones_k = jnp.ones((K,), jnp.float32)

    deg = _deg_kernel(src, dst, zeros1, ones_k)
    degT = deg.reshape(2 * NC, N).T

    xp, nsrc, ndst = pl.pallas_call(
        _norm_body,
        grid=(GRID,),
        in_specs=[
            pl.BlockSpec((RB, 2 * NC), lambda i: (i, 0)),
            pl.BlockSpec((RB, D), lambda i: (i, 0)),
        ],
        out_specs=[
            pl.BlockSpec((2, RB, DH), lambda i: (0, i, 0)),
            pl.BlockSpec((RB, 1), lambda i: (i, 0)),
            pl.BlockSpec((RB, 1), lambda i: (i, 0)),
        ],
        out_shape=[
            jax.ShapeDtypeStruct((2, N, DH), jnp.float32),
            jax.ShapeDtypeStruct((N, 1), jnp.float32),
            jax.ShapeDtypeStruct((N, 1), jnp.float32),
        ],
    )(degT, x)

    agg1 = _agg_kernel(xp.reshape(2 * N, DH), src, dst, zeros2)

    h1p = pl.pallas_call(
        _layer1_body,
        grid=(GRID,),
        in_specs=[
            pl.BlockSpec((1, RB, DH), lambda i: (0, i, 0)),
            pl.BlockSpec((1, RB, DH), lambda i: (1, i, 0)),
            pl.BlockSpec((DH, H), lambda i: (0, 0)),
            pl.BlockSpec((DH, H), lambda i: (1, 0)),
            pl.BlockSpec((1, H), lambda i: (0, 0)),
            pl.BlockSpec((RB, 1), lambda i: (i, 0)),
            pl.BlockSpec((RB, 1), lambda i: (i, 0)),
        ],
        out_specs=pl.BlockSpec((2, RB, DH), lambda i: (0, i, 0)),
        out_shape=jax.ShapeDtypeStruct((2, N, DH), jnp.float32),
    )(agg1, agg1, W1, W1, b1.reshape(1, H), ndst, nsrc)

    agg2 = _agg_kernel(h1p.reshape(2 * N, DH), src, dst, zeros2)

    out = pl.pallas_call(
        _final_body,
        grid=(GRID,),
        in_specs=[
            pl.BlockSpec((1, RB, DH), lambda i: (0, i, 0)),
            pl.BlockSpec((1, RB, DH), lambda i: (1, i, 0)),
            pl.BlockSpec((DH, H), lambda i: (0, 0)),
            pl.BlockSpec((DH, H), lambda i: (1, 0)),
            pl.BlockSpec((1, H), lambda i: (0, 0)),
            pl.BlockSpec((RB, 1), lambda i: (i, 0)),
            pl.BlockSpec((H, H), lambda i: (0, 0)),
            pl.BlockSpec((1, H), lambda i: (0, 0)),
            pl.BlockSpec((H, H), lambda i: (0, 0)),
            pl.BlockSpec((1, H), lambda i: (0, 0)),
            pl.BlockSpec((H, 1), lambda i: (0, 0)),
            pl.BlockSpec((1, 1), lambda i: (0, 0)),
        ],
        out_specs=pl.BlockSpec((1, 1), lambda i: (0, 0)),
        out_shape=jax.ShapeDtypeStruct((1, 1), jnp.float32),
        scratch_shapes=[pltpu.VMEM((1, H), jnp.float32)],
    )(agg2, agg2, W2, W2, b2.reshape(1, H), ndst,
      Wc1, bc1.reshape(1, H), Wc2, bc2.reshape(1, H),
      Wc3, bc3.reshape(1, 1))

    return out


# speedup vs baseline: 9.1875x; 1.0368x over previous
import functools

import jax
import jax.numpy as jnp
from jax import lax
from jax.experimental import pallas as pl
from jax.experimental.pallas import tpu as pltpu
from jax.experimental.pallas import tpu_sc as plsc

N = 10000
E = 320000
D = 128
H = 128
DH = D // 2

NC = 2
NS = 16
EPT = E // NS
K = 125
NITER = EPT // K
SEG = 4
SROWS = NITER // SEG
DNITER = E // (NC * NS) // K
EROWS = E // K

RPT = 624
RTAIL = N - NS * RPT

_MESH = plsc.VectorSubcoreMesh(core_axis_name="c", subcore_axis_name="s")


@functools.partial(
    pl.kernel,
    out_type=jax.ShapeDtypeStruct((NC * 2 * N,), jnp.float32),
    mesh=_MESH,
    scratch_types=[
        pltpu.VMEM((DNITER, K), jnp.int32),
        pltpu.VMEM((DNITER, K), jnp.int32),
        pltpu.VMEM((K,), jnp.float32),
        pltpu.VMEM((RPT,), jnp.float32),
        pltpu.VMEM_SHARED((N,), jnp.float32),
        pltpu.VMEM_SHARED((N,), jnp.float32),
        pltpu.SemaphoreType.DMA,
    ],
    compiler_params=pltpu.CompilerParams(use_tc_tiling_on_sc=False),
)
def _deg_kernel(src_hbm, dst_hbm, zeros1_hbm, ones_hbm, out_hbm,
                src_all, dst_all, ones_v, stage_v, acc_src, acc_dst, sem):
    cid = lax.axis_index("c")
    sid = lax.axis_index("s")
    wid = cid * NS + sid

    pltpu.sync_copy(src_hbm.at[pl.ds(wid * DNITER, DNITER)], src_all)
    pltpu.sync_copy(dst_hbm.at[pl.ds(wid * DNITER, DNITER)], dst_all)

    pltpu.sync_copy(zeros1_hbm, stage_v)
    pltpu.sync_copy(stage_v, acc_src.at[pl.ds(sid * RPT, RPT)])
    pltpu.sync_copy(stage_v, acc_dst.at[pl.ds(sid * RPT, RPT)])

    @pl.when(sid == NS - 1)
    def _zero_tail():
        pltpu.sync_copy(stage_v.at[pl.ds(0, RTAIL)],
                        acc_src.at[pl.ds(NS * RPT, RTAIL)])
        pltpu.sync_copy(stage_v.at[pl.ds(0, RTAIL)],
                        acc_dst.at[pl.ds(NS * RPT, RTAIL)])

    pltpu.sync_copy(ones_hbm, ones_v)
    plsc.subcore_barrier()

    def body(i, carry):
        pltpu.sync_copy(ones_v, acc_src.at[src_all.at[i]], add=True)
        pltpu.sync_copy(ones_v, acc_dst.at[dst_all.at[i]], add=True)
        return carry

    lax.fori_loop(0, DNITER, body, 0)
    plsc.subcore_barrier()

    obase = cid * 2 * N
    pltpu.sync_copy(acc_src.at[pl.ds(sid * RPT, RPT)], stage_v)
    pltpu.sync_copy(stage_v, out_hbm.at[pl.ds(obase + sid * RPT, RPT)])
    pltpu.sync_copy(acc_dst.at[pl.ds(sid * RPT, RPT)], stage_v)
    pltpu.sync_copy(stage_v, out_hbm.at[pl.ds(obase + N + sid * RPT, RPT)])

    @pl.when(sid == NS - 1)
    def _write_tail():
        pltpu.sync_copy(acc_src.at[pl.ds(NS * RPT, RTAIL)],
                        stage_v.at[pl.ds(0, RTAIL)])
        pltpu.sync_copy(stage_v.at[pl.ds(0, RTAIL)],
                        out_hbm.at[pl.ds(obase + NS * RPT, RTAIL)])
        pltpu.sync_copy(acc_dst.at[pl.ds(NS * RPT, RTAIL)],
                        stage_v.at[pl.ds(0, RTAIL)])
        pltpu.sync_copy(stage_v.at[pl.ds(0, RTAIL)],
                        out_hbm.at[pl.ds(obase + N + NS * RPT, RTAIL)])


@functools.partial(
    pl.kernel,
    out_type=jax.ShapeDtypeStruct((NC, N, DH), jnp.float32),
    mesh=_MESH,
    scratch_types=[
        pltpu.VMEM((SROWS, K), jnp.int32),
        pltpu.VMEM((SROWS, K), jnp.int32),
        pltpu.VMEM((K, DH), jnp.float32),
        pltpu.VMEM((K, DH), jnp.float32),
        pltpu.VMEM((K, DH), jnp.float32),
        pltpu.VMEM((K, DH), jnp.float32),
        pltpu.VMEM_SHARED((N, DH), jnp.float32),
        pltpu.SemaphoreType.DMA,
        pltpu.SemaphoreType.DMA,
        pltpu.SemaphoreType.DMA,
        pltpu.SemaphoreType.DMA,
        pltpu.SemaphoreType.DMA,
        pltpu.SemaphoreType.DMA,
        pltpu.SemaphoreType.DMA,
        pltpu.SemaphoreType.DMA,
    ],
    compiler_params=pltpu.CompilerParams(use_tc_tiling_on_sc=False),
)
def _agg_kernel(feat_hbm, src_hbm, dst_hbm, zeros2_hbm, out_hbm,
                src_seg, dst_seg, rows0, rows1, rows2, rows3, acc_sh,
                gs0, gs1, gs2, gs3, ss0, ss1, ss2, ss3):
    cid = lax.axis_index("c")
    sid = lax.axis_index("s")
    feat = feat_hbm.at[pl.ds(cid * N, N)]

    pltpu.sync_copy(zeros2_hbm, rows0)
    for j in range(RPT // K):
        pltpu.sync_copy(rows0, acc_sh.at[pl.ds(sid * RPT + j * K, K)])
    pltpu.sync_copy(rows0.at[pl.ds(0, RPT % K)],
                    acc_sh.at[pl.ds(sid * RPT + (RPT // K) * K, RPT % K)])

    @pl.when(sid == NS - 1)
    def _zero_tail():
        pltpu.sync_copy(rows0.at[pl.ds(0, RTAIL)],
                        acc_sh.at[pl.ds(NS * RPT, RTAIL)])

    plsc.subcore_barrier()

    bufs = (rows0, rows1, rows2, rows3)
    gsems = (gs0, gs1, gs2, gs3)
    ssems = (ss0, ss1, ss2, ss3)

    def wait_gather(i, b):
        pltpu.make_async_copy(feat.at[src_seg.at[i]], bufs[b],
                              gsems[b]).wait()

    def wait_scatter(b):
        pltpu.make_async_copy(bufs[b], acc_sh.at[dst_seg.at[0]],
                              ssems[b]).wait()

    def seg_body(s, carry):
        base_row = sid * NITER + s * SROWS
        pltpu.sync_copy(src_hbm.at[pl.ds(base_row, SROWS)], src_seg)
        pltpu.sync_copy(dst_hbm.at[pl.ds(base_row, SROWS)], dst_seg)
        pltpu.async_copy(feat.at[src_seg.at[0]], rows0, gs0)
        pltpu.async_copy(feat.at[src_seg.at[1]], rows1, gs1)

        def body(g, c):
            for b in range(4):
                i = g * 4 + b
                wait_gather(i, b)
                pltpu.async_copy(bufs[b], acc_sh.at[dst_seg.at[i]],
                                 ssems[b], add=True)
                b2 = (b + 2) % 4

                @pl.when(i + 2 < SROWS)
                def _prefetch():
                    @pl.when(i >= 2)
                    def _recycle():
                        wait_scatter(b2)

                    pltpu.async_copy(feat.at[src_seg.at[i + 2]], bufs[b2],
                                     gsems[b2])
            return c

        lax.fori_loop(0, SROWS // 4, body, 0)
        for b in range(4):
            wait_scatter(b)
        return carry

    lax.fori_loop(0, SEG, seg_body, 0)
    plsc.subcore_barrier()

    for j in range(RPT // K):
        pltpu.sync_copy(acc_sh.at[pl.ds(sid * RPT + j * K, K)], rows0)
        pltpu.sync_copy(rows0, out_hbm.at[cid, pl.ds(sid * RPT + j * K, K)])
    pltpu.sync_copy(acc_sh.at[pl.ds(sid * RPT + (RPT // K) * K, RPT % K)],
                    rows0.at[pl.ds(0, RPT % K)])
    pltpu.sync_copy(rows0.at[pl.ds(0, RPT % K)],
                    out_hbm.at[cid, pl.ds(sid * RPT + (RPT // K) * K, RPT % K)])

    @pl.when(sid == NS - 1)
    def _write_tail():
        pltpu.sync_copy(acc_sh.at[pl.ds(NS * RPT, RTAIL)],
                        rows1.at[pl.ds(0, RTAIL)])
        pltpu.sync_copy(rows1.at[pl.ds(0, RTAIL)],
                        out_hbm.at[cid, pl.ds(NS * RPT, RTAIL)])


RB = 1000
GRID = N // RB


def _norm_body(deg_ref, x_ref, xp_ref, ns_ref, nd_ref):
    deg = deg_ref[...]
    dsrc = deg[:, 0:1] + deg[:, 2:3]
    ddst = deg[:, 1:2] + deg[:, 3:4]
    ns = lax.rsqrt(jnp.where(dsrc > 0, dsrc, 1.0))
    nd = lax.rsqrt(jnp.where(ddst > 0, ddst, 1.0))
    xp = x_ref[...] * ns
    xp_ref[0] = xp[:, :DH]
    xp_ref[1] = xp[:, DH:]
    ns_ref[...] = ns
    nd_ref[...] = nd


def _layer1_body(a0_ref, a1_ref, wa_ref, wb_ref, b_ref, nd_ref, ns_ref,
                 out_ref):
    h = (jnp.dot(a0_ref[0], wa_ref[...], preferred_element_type=jnp.float32, precision=lax.Precision.HIGHEST)
         + jnp.dot(a1_ref[0], wb_ref[...], preferred_element_type=jnp.float32, precision=lax.Precision.HIGHEST))
    h = jnp.maximum(h * nd_ref[...] + b_ref[...], 0.0)
    h = h * ns_ref[...]
    out_ref[0] = h[:, :DH]
    out_ref[1] = h[:, DH:]


def _final_body(a0_ref, a1_ref, wa_ref, wb_ref, b_ref, nd_ref,
                wc1_ref, bc1_ref, wc2_ref, bc2_ref, wc3_ref, bc3_ref,
                out_ref, acc_ref):
    i = pl.program_id(0)

    @pl.when(i == 0)
    def _init():
        acc_ref[...] = jnp.zeros_like(acc_ref)

    h = (jnp.dot(a0_ref[0], wa_ref[...], preferred_element_type=jnp.float32, precision=lax.Precision.HIGHEST)
         + jnp.dot(a1_ref[0], wb_ref[...], preferred_element_type=jnp.float32, precision=lax.Precision.HIGHEST))
    h = jnp.maximum(h * nd_ref[...] + b_ref[...], 0.0)
    acc_ref[...] += jnp.sum(h, axis=0, keepdims=True)

    @pl.when(i == pl.num_programs(0) - 1)
    def _head():
        hg = acc_ref[...] * (1.0 / N)
        o = jnp.dot(hg, wc1_ref[...], preferred_element_type=jnp.float32, precision=lax.Precision.HIGHEST)
        o = jnp.maximum(o + bc1_ref[...], 0.0)
        o = jnp.dot(o, wc2_ref[...], preferred_element_type=jnp.float32, precision=lax.Precision.HIGHEST)
        o = jnp.maximum(o + bc2_ref[...], 0.0)
        out_ref[...] = (jnp.dot(o, wc3_ref[...], preferred_element_type=jnp.float32, precision=lax.Precision.HIGHEST)
                        + bc3_ref[...])


def kernel(x, edge_index, W1, b1, W2, b2, Wc1, bc1, Wc2, bc2, Wc3, bc3):
    edge_index = edge_index.astype(jnp.int32)
    src = edge_index[0].reshape(EROWS, K)
    dst = edge_index[1].reshape(EROWS, K)
    zeros1 = jnp.zeros((RPT,), jnp.float32)
    zeros2 = jnp.zeros((K, DH), jnp.float32)
    ones_k = jnp.ones((K,), jnp.float32)

    deg = _deg_kernel(src, dst, zeros1, ones_k)
    degT = deg.reshape(2 * NC, N).T

    xp, nsrc, ndst = pl.pallas_call(
        _norm_body,
        grid=(GRID,),
        in_specs=[
            pl.BlockSpec((RB, 2 * NC), lambda i: (i, 0)),
            pl.BlockSpec((RB, D), lambda i: (i, 0)),
        ],
        out_specs=[
            pl.BlockSpec((2, RB, DH), lambda i: (0, i, 0)),
            pl.BlockSpec((RB, 1), lambda i: (i, 0)),
            pl.BlockSpec((RB, 1), lambda i: (i, 0)),
        ],
        out_shape=[
            jax.ShapeDtypeStruct((2, N, DH), jnp.float32),
            jax.ShapeDtypeStruct((N, 1), jnp.float32),
            jax.ShapeDtypeStruct((N, 1), jnp.float32),
        ],
    )(degT, x)

    agg1 = _agg_kernel(xp.reshape(2 * N, DH), src, dst, zeros2)

    h1p = pl.pallas_call(
        _layer1_body,
        grid=(GRID,),
        in_specs=[
            pl.BlockSpec((1, RB, DH), lambda i: (0, i, 0)),
            pl.BlockSpec((1, RB, DH), lambda i: (1, i, 0)),
            pl.BlockSpec((DH, H), lambda i: (0, 0)),
            pl.BlockSpec((DH, H), lambda i: (1, 0)),
            pl.BlockSpec((1, H), lambda i: (0, 0)),
            pl.BlockSpec((RB, 1), lambda i: (i, 0)),
            pl.BlockSpec((RB, 1), lambda i: (i, 0)),
        ],
        out_specs=pl.BlockSpec((2, RB, DH), lambda i: (0, i, 0)),
        out_shape=jax.ShapeDtypeStruct((2, N, DH), jnp.float32),
    )(agg1, agg1, W1, W1, b1.reshape(1, H), ndst, nsrc)

    agg2 = _agg_kernel(h1p.reshape(2 * N, DH), src, dst, zeros2)

    out = pl.pallas_call(
        _final_body,
        grid=(GRID,),
        in_specs=[
            pl.BlockSpec((1, RB, DH), lambda i: (0, i, 0)),
            pl.BlockSpec((1, RB, DH), lambda i: (1, i, 0)),
            pl.BlockSpec((DH, H), lambda i: (0, 0)),
            pl.BlockSpec((DH, H), lambda i: (1, 0)),
            pl.BlockSpec((1, H), lambda i: (0, 0)),
            pl.BlockSpec((RB, 1), lambda i: (i, 0)),
            pl.BlockSpec((H, H), lambda i: (0, 0)),
            pl.BlockSpec((1, H), lambda i: (0, 0)),
            pl.BlockSpec((H, H), lambda i: (0, 0)),
            pl.BlockSpec((1, H), lambda i: (0, 0)),
            pl.BlockSpec((H, 1), lambda i: (0, 0)),
            pl.BlockSpec((1, 1), lambda i: (0, 0)),
        ],
        out_specs=pl.BlockSpec((1, 1), lambda i: (0, 0)),
        out_shape=jax.ShapeDtypeStruct((1, 1), jnp.float32),
        scratch_shapes=[pltpu.VMEM((1, H), jnp.float32)],
    )(agg2, agg2, W2, W2, b2.reshape(1, H), ndst,
      Wc1, bc1.reshape(1, H), Wc2, bc2.reshape(1, H),
      Wc3, bc3.reshape(1, 1))

    return out
